# Initial kernel scaffold; baseline (speedup 1.0000x reference)
import jax
import jax.numpy as jnp
from jax.experimental import pallas as pl


def kernel(sentence1, sentence2, table, W1, b1, W2, b2):
    raise NotImplementedError("write your pallas kernel here")


# speedup vs baseline: 1.5339x; 1.5339x over previous
import functools

import jax
import jax.numpy as jnp
from jax import lax
from jax.experimental import pallas as pl
from jax.experimental.pallas import tpu as pltpu
from jax.experimental.pallas import tpu_sc as plsc

B = 4096
L = 50
D = 64
NC = 2
NS = 16
NW = NC * NS

ROWS_TOTAL = 2 * B
ROWS_PER_W = ROWS_TOTAL // NW
SEGS_PER_CHUNK = 2
CHUNK = SEGS_PER_CHUNK * L
NCHUNK = ROWS_PER_W // SEGS_PER_CHUNK
NBUF = 2


def _sc_pool_body(idx_hbm, table_hbm, out_hbm, idx_v, rows_v, stage_v, sems):
    c = lax.axis_index("c")
    s = lax.axis_index("s")
    wid = s * NC + c

    pltpu.sync_copy(idx_hbm.at[wid], idx_v)

    def start_gather(ch, buf):
        return pltpu.async_copy(
            table_hbm.at[idx_v.at[ch]], rows_v.at[buf], sems.at[buf]
        )

    for b in range(NBUF):
        start_gather(b, b)

    def chunk_body(ch, carry):
        buf = lax.rem(ch, NBUF)
        pltpu.make_async_copy(
            table_hbm.at[idx_v.at[ch]], rows_v.at[buf], sems.at[buf]
        ).wait()

        for seg in range(SEGS_PER_CHUNK):
            def row_body(r, accs):
                base = seg * L + r
                return tuple(
                    accs[k] + rows_v[buf, base, pl.ds(k * 16, 16)]
                    for k in range(D // 16)
                )

            zeros = tuple(
                jnp.zeros((16,), jnp.float32) for _ in range(D // 16)
            )
            accs = lax.fori_loop(0, L, row_body, zeros, unroll=5)
            for k in range(D // 16):
                stage_v[SEGS_PER_CHUNK * ch + seg, pl.ds(k * 16, 16)] = (
                    accs[k] * (1.0 / L)
                )

        @pl.when(ch + NBUF < NCHUNK)
        def _():
            start_gather(ch + NBUF, buf)

        return carry

    lax.fori_loop(0, NCHUNK, chunk_body, 0)

    pltpu.sync_copy(stage_v, out_hbm.at[pl.ds(wid * ROWS_PER_W, ROWS_PER_W)])


@jax.jit
def _sc_pool(idx, table):
    mesh = plsc.VectorSubcoreMesh(core_axis_name="c", subcore_axis_name="s")
    return pl.kernel(
        _sc_pool_body,
        out_type=jax.ShapeDtypeStruct((ROWS_TOTAL, D), jnp.float32),
        mesh=mesh,
        scratch_types=[
            pltpu.VMEM((NCHUNK, CHUNK), jnp.int32),
            pltpu.VMEM((NBUF, CHUNK, D), jnp.float32),
            pltpu.VMEM((ROWS_PER_W, D), jnp.float32),
            pltpu.SemaphoreType.DMA((NBUF,)),
        ],
        compiler_params=pltpu.CompilerParams(use_tc_tiling_on_sc=False),
    )(idx, table)


def _tc_mlp_body(u_ref, v_ref, w1t_ref, b1_ref, w2t_ref, b2_ref, out_ref):
    u = u_ref[...]
    v = v_ref[...]
    combined = jnp.concatenate([u, v, jnp.abs(u - v), u * v], axis=1)
    h = jnp.dot(combined, w1t_ref[...], preferred_element_type=jnp.float32)
    h = jnp.maximum(h + b1_ref[...], 0.0)
    logits = jnp.dot(h, w2t_ref[...], preferred_element_type=jnp.float32)
    logits = logits + b2_ref[...]
    m = jnp.max(logits, axis=1, keepdims=True)
    e = jnp.exp(logits - m)
    out_ref[...] = e / jnp.sum(e, axis=1, keepdims=True)


@jax.jit
def _tc_mlp(u, v, w1t, b1, w2t, b2):
    return pl.pallas_call(
        _tc_mlp_body,
        out_shape=jax.ShapeDtypeStruct((B, w2t.shape[1]), jnp.float32),
    )(u, v, w1t, b1, w2t, b2)


@jax.jit
def kernel(sentence1, sentence2, table, W1, b1, W2, b2):
    idx = jnp.concatenate(
        [sentence1.reshape(-1), sentence2.reshape(-1)]
    ).reshape(NW, NCHUNK, CHUNK)
    uv = _sc_pool(idx, table)
    u = uv[:B]
    v = uv[B:]
    nl = W2.shape[0]
    out = _tc_mlp(
        u, v, W1.T, b1.reshape(1, -1), W2.T, b2.reshape(1, -1)
    )
    return out[:, :nl]
